# baseline (speedup 1.0000x reference)
import functools

import jax
import jax.numpy as jnp
from jax import lax
from jax.experimental import pallas as pl
from jax.experimental.pallas import tpu as pltpu
from jax.experimental.pallas import tpu_sc as plsc

NW = 32
SC_MESH = dict(core_axis_name="c", subcore_axis_name="s")

ACT_DIM = 768
DICT_SIZE = 24576
K = 64
EXPERTS = 16
E_ROUTED = 2
EXP_DICT = DICT_SIZE // EXPERTS
N_TOK = 8192

BM = 128
M_PAD = 16384 + EXPERTS * BM
NBLK = M_PAD // BM
GATE_BM = 256

_HI = jax.lax.Precision.DEFAULT


def _gate_body(x_ref, wt_ref, gb_ref, bg_ref, eidx_ref, w2_ref):
    xb = x_ref[...] - bg_ref[...]
    logits = jax.lax.dot_general(
        xb, wt_ref[...], (((1,), (0,)), ((), ())),
        preferred_element_type=jnp.float32, precision=_HI) + gb_ref[...]
    p = jax.nn.softmax(logits, axis=-1)
    iota = jax.lax.broadcasted_iota(jnp.int32, p.shape, 1)
    m1 = jnp.max(p, axis=1, keepdims=True)
    a1 = jnp.min(jnp.where(p >= m1, iota, EXPERTS), axis=1, keepdims=True)
    p2 = jnp.where(iota == a1, -jnp.inf, p)
    m2 = jnp.max(p2, axis=1, keepdims=True)
    a2 = jnp.min(jnp.where(p2 >= m2, iota, EXPERTS), axis=1, keepdims=True)
    d = m2 - m1
    ed = jnp.exp(d)
    w1 = 1.0 / (1.0 + ed)
    w2 = ed / (1.0 + ed)
    eidx_ref[...] = jnp.concatenate([a1, a2], axis=1)
    w2_ref[...] = jnp.concatenate([w1, w2], axis=1)


def _gating(x, gate_W, gate_b, b_gate):
    grid = (N_TOK // GATE_BM,)
    return pl.pallas_call(
        _gate_body,
        grid=grid,
        in_specs=[
            pl.BlockSpec((GATE_BM, ACT_DIM), lambda i: (i, 0)),
            pl.BlockSpec((ACT_DIM, EXPERTS), lambda i: (0, 0)),
            pl.BlockSpec((1, EXPERTS), lambda i: (0, 0)),
            pl.BlockSpec((1, ACT_DIM), lambda i: (0, 0)),
        ],
        out_specs=[
            pl.BlockSpec((GATE_BM, E_ROUTED), lambda i: (i, 0)),
            pl.BlockSpec((GATE_BM, E_ROUTED), lambda i: (i, 0)),
        ],
        out_shape=[
            jax.ShapeDtypeStruct((N_TOK, E_ROUTED), jnp.int32),
            jax.ShapeDtypeStruct((N_TOK, E_ROUTED), jnp.float32),
        ],
    )(x, gate_W.T, gate_b[None, :], b_gate[None, :])


def _dispatch(eidx, w2):
    e_flat = eidx.reshape(-1)
    w_flat = w2.reshape(-1)
    onehot = (e_flat[:, None] ==
              jnp.arange(EXPERTS, dtype=e_flat.dtype)[None, :]).astype(jnp.int32)
    csum = jnp.cumsum(onehot, axis=0)
    rank = jnp.sum(onehot * csum, axis=1) - 1
    counts = csum[-1]
    padded = ((counts + BM - 1) // BM) * BM
    seg_pad = jnp.concatenate([jnp.zeros((1,), jnp.int32),
                               jnp.cumsum(padded)[:-1].astype(jnp.int32)])
    dest = seg_pad[e_flat] + rank
    ar = jnp.arange(e_flat.shape[0], dtype=jnp.int32)
    tok_ids = jnp.zeros((M_PAD,), jnp.int32).at[dest].set(ar // E_ROUTED)
    w_sorted = jnp.zeros((M_PAD,), jnp.float32).at[dest].set(w_flat)
    blk_expert = jnp.searchsorted(
        jnp.cumsum(padded), jnp.arange(NBLK, dtype=jnp.int32) * BM,
        side='right').astype(jnp.int32)
    blk_expert = jnp.minimum(blk_expert, EXPERTS - 1)
    return tok_ids, w_sorted, dest, blk_expert


def _gemm_body(be_ref, xs_ref, w_ref, encw_ref, encb_ref, bdec_ref, zs_ref):
    xb = xs_ref[...] - bdec_ref[...]
    z = jax.lax.dot_general(
        xb, encw_ref[...], (((1,), (1,)), ((), ())),
        preferred_element_type=jnp.float32, precision=_HI)
    z = jnp.maximum(z + encb_ref[...], 0.0) * w_ref[...]
    zs_ref[...] = z


def _grouped_gemm(xs, w_sorted, blk_expert, enc_W, enc_b, b_dec):
    grid_spec = pltpu.PrefetchScalarGridSpec(
        num_scalar_prefetch=1,
        grid=(NBLK,),
        in_specs=[
            pl.BlockSpec((BM, ACT_DIM), lambda i, be: (i, 0)),
            pl.BlockSpec((BM, 1), lambda i, be: (i, 0)),
            pl.BlockSpec((None, EXP_DICT, ACT_DIM), lambda i, be: (be[i], 0, 0)),
            pl.BlockSpec((None, 1, EXP_DICT), lambda i, be: (be[i], 0, 0)),
            pl.BlockSpec((1, ACT_DIM), lambda i, be: (0, 0)),
        ],
        out_specs=pl.BlockSpec((BM, EXP_DICT), lambda i, be: (i, 0)),
    )
    return pl.pallas_call(
        _gemm_body,
        grid_spec=grid_spec,
        out_shape=jax.ShapeDtypeStruct((M_PAD, EXP_DICT), jnp.float32),
    )(blk_expert, xs, w_sorted[:, None], enc_W, enc_b[:, None, :],
      b_dec[None, :])


THR_BM = 256


def _thr_body(cs_ref, thr_ref):
    v = cs_ref[...]
    lo = jnp.full((THR_BM, 1), -1, jnp.int32)
    hi = jnp.full((THR_BM, 1), 0x7F7FFFFF, jnp.int32)

    def step(_, carry):
        lo, hi = carry
        mid = lo + (hi - lo) // 2
        cnt = jnp.sum((v > mid).astype(jnp.int32), axis=1, keepdims=True)
        small = cnt < K
        return jnp.where(small, lo, mid + 1), jnp.where(small, mid, hi)

    lo, hi = jax.lax.fori_loop(0, 31, step, (lo, hi))
    thr_ref[...] = hi.reshape(1, THR_BM)


def _thresholds(cs_bits):
    n = N_TOK // THR_BM
    out = pl.pallas_call(
        _thr_body,
        grid=(n,),
        in_specs=[pl.BlockSpec((THR_BM, E_ROUTED * EXP_DICT), lambda i: (i, 0))],
        out_specs=pl.BlockSpec((None, 1, THR_BM), lambda i: (i, 0, 0)),
        out_shape=jax.ShapeDtypeStruct((n, 1, THR_BM), jnp.int32),
    )(cs_bits)
    return out.reshape(N_TOK)


def _sc_gather_rows(table, idx, D, CH):
    B = idx.shape[0]
    bpw = B // NW
    nch = bpw // CH
    mesh = plsc.VectorSubcoreMesh(**SC_MESH)

    @functools.partial(
        pl.kernel,
        out_type=jax.ShapeDtypeStruct((B, D), table.dtype),
        mesh=mesh,
        scratch_types=[
            pltpu.VMEM((bpw,), jnp.int32),
            pltpu.VMEM((CH, D), table.dtype),
            pltpu.SemaphoreType.DMA,
        ],
        compiler_params=pltpu.CompilerParams(needs_layout_passes=False),
    )
    def k(table_h, idx_h, out_h, idx_v, buf_v, sem):
        wid = lax.axis_index("s") * 2 + lax.axis_index("c")
        base = wid * bpw
        pltpu.sync_copy(idx_h.at[pl.ds(base, bpw)], idx_v)

        def chunk(c, carry):
            pltpu.async_copy(
                table_h.at[idx_v.at[pl.ds(c * CH, CH)]], buf_v, sem).wait()
            pltpu.sync_copy(buf_v, out_h.at[pl.ds(base + c * CH, CH)])
            return carry

        lax.fori_loop(0, nch, chunk, 0)

    return k(table, idx)


TPW = N_TOK // NW
NCH = (E_ROUTED * EXP_DICT) // 16
ND = ACT_DIM // 16


def _sc_select_decode(cs, thr, base_flat, decoder, b_dec):
    mesh = plsc.VectorSubcoreMesh(**SC_MESH)

    @functools.partial(
        pl.kernel,
        out_type=jax.ShapeDtypeStruct((N_TOK, ACT_DIM), jnp.float32),
        mesh=mesh,
        scratch_types=[
            pltpu.VMEM((2, E_ROUTED * EXP_DICT), jnp.float32),
            pltpu.VMEM((TPW,), jnp.int32),
            pltpu.VMEM((TPW * E_ROUTED,), jnp.int32),
            pltpu.VMEM((2, K), jnp.float32),
            pltpu.VMEM((2, K), jnp.int32),
            pltpu.VMEM((2, K, ACT_DIM), jnp.float32),
            pltpu.VMEM((ACT_DIM,), jnp.float32),
            pltpu.VMEM((ACT_DIM,), jnp.float32),
            pltpu.SemaphoreType.DMA((2,)),
            pltpu.SemaphoreType.DMA((2,)),
        ],
        compiler_params=pltpu.CompilerParams(needs_layout_passes=False),
    )
    def k(cs_h, thr_h, base_h, dec_h, bdec_h, out_h, csv, thrv, basev, valsv,
          idxsv, rowsv, bdecv, outv, cssem, rowsem):
        wid = lax.axis_index("s") * 2 + lax.axis_index("c")
        pltpu.sync_copy(thr_h.at[pl.ds(wid * TPW, TPW)], thrv)
        pltpu.sync_copy(base_h.at[pl.ds(wid * TPW * E_ROUTED,
                                        TPW * E_ROUTED)], basev)
        pltpu.sync_copy(bdec_h, bdecv)
        lanes = lax.iota(jnp.int32, 16)

        def cs_dma(j, s):
            return pltpu.make_async_copy(
                cs_h.at[wid * TPW + j], csv.at[s], cssem.at[s])

        def row_dma(s):
            return pltpu.make_async_copy(
                dec_h.at[idxsv.at[s]], rowsv.at[s], rowsem.at[s])

        def select(j, s):
            for b in range(K // 16):
                valsv[s, pl.ds(b * 16, 16)] = jnp.zeros((16,), jnp.float32)
                idxsv[s, pl.ds(b * 16, 16)] = jnp.zeros((16,), jnp.int32)
            tvec = plsc.load_gather(thrv, [jnp.zeros((16,), jnp.int32) + j])
            b0 = plsc.load_gather(basev, [jnp.zeros((16,), jnp.int32) + 2 * j])
            b1 = plsc.load_gather(basev,
                                  [jnp.zeros((16,), jnp.int32) + 2 * j + 1])

            def group(g, off):
                vfs, vis, masks, cnts, css = [], [], [], [], []
                for u in range(4):
                    vf = csv[s, pl.ds(g * 64 + u * 16, 16)]
                    vi = plsc.bitcast(vf, jnp.int32)
                    mask = (vi >= tvec) & (vi > 0)
                    vfs.append(vf)
                    vis.append(vi)
                    masks.append(mask)
                    cnts.append(jnp.sum(mask.astype(jnp.int32)))
                    css.append(plsc.cumsum(mask.astype(jnp.int32)))

                @pl.when(off < K)
                def _():
                    o = off
                    for u in range(4):
                        c = g * 4 + u
                        sh = (c >= NCH // 2).astype(jnp.int32)
                        gidx = b0 + sh * (b1 - b0) + (
                            lanes + c * 16 - sh * (EXP_DICT * E_ROUTED // 2))
                        pos = o + css[u] - 1
                        mask2 = masks[u] & (pos < K)
                        plsc.store_scatter(valsv.at[s], [pos], vfs[u],
                                           mask=mask2)
                        plsc.store_scatter(idxsv.at[s], [pos], gidx,
                                           mask=mask2)
                        o = o + cnts[u]

                return off + cnts[0] + cnts[1] + cnts[2] + cnts[3]

            lax.fori_loop(0, NCH // 4, group, jnp.int32(0))

        def accum_store(j, s):
            def accum(kk, acc):
                ws = [plsc.load_gather(valsv.at[s],
                                       [jnp.zeros((16,), jnp.int32)
                                        + (4 * kk + u)])
                      for u in range(4)]
                for u in range(4):
                    acc = tuple(acc[d] + ws[u] *
                                rowsv[s, 4 * kk + u, pl.ds(d * 16, 16)]
                                for d in range(ND))
                return acc

            acc0 = tuple(bdecv[pl.ds(d * 16, 16)] for d in range(ND))
            acc = lax.fori_loop(0, K // 4, accum, acc0)
            for d in range(ND):
                outv[pl.ds(d * 16, 16)] = acc[d]
            pltpu.sync_copy(outv, out_h.at[wid * TPW + j])

        def phase(j, s):
            cs_dma(j, s).wait()
            select(j, s)
            row_dma(s).start()

            @pl.when(j >= 1)
            def _():
                row_dma(1 - s).wait()
                accum_store(j - 1, 1 - s)

            @pl.when(j + 2 < TPW)
            def _():
                cs_dma(j + 2, s).start()

        cs_dma(0, 0).start()
        cs_dma(1, 1).start()

        def pair(jj, carry):
            phase(2 * jj, 0)
            phase(2 * jj + 1, 1)
            return carry

        lax.fori_loop(0, TPW // 2, pair, 0)
        row_dma(1).wait()
        accum_store(TPW - 1, 1)

    return k(cs, thr, base_flat, decoder, b_dec)


def kernel(x, gate_W, gate_b, enc_W, enc_b, decoder, b_gate, b_dec):
    eidx, w2 = _gating(x, gate_W, gate_b, b_gate)
    tok_ids, w_sorted, pos_flat, blk_expert = _dispatch(eidx, w2)
    xs = _sc_gather_rows(x, tok_ids, ACT_DIM, 96)
    zs = _grouped_gemm(xs, w_sorted, blk_expert, enc_W, enc_b, b_dec)
    cs = _sc_gather_rows(zs, pos_flat, EXP_DICT, 64)
    cs = cs.reshape(N_TOK, E_ROUTED * EXP_DICT)
    thr = _thresholds(cs.view(jnp.int32))
    base_flat = (eidx * EXP_DICT).reshape(-1)
    return _sc_select_decode(cs, thr, base_flat, decoder, b_dec)


# speedup vs baseline: 1.2364x; 1.2364x over previous
import functools

import jax
import jax.numpy as jnp
from jax import lax
from jax.experimental import pallas as pl
from jax.experimental.pallas import tpu as pltpu
from jax.experimental.pallas import tpu_sc as plsc

NW = 32
SC_MESH = dict(core_axis_name="c", subcore_axis_name="s")

ACT_DIM = 768
DICT_SIZE = 24576
K = 64
EXPERTS = 16
E_ROUTED = 2
EXP_DICT = DICT_SIZE // EXPERTS
N_TOK = 8192

BM = 128
M_PAD = 16384 + EXPERTS * BM
NBLK = M_PAD // BM
GATE_BM = 256

_HI = jax.lax.Precision.DEFAULT


def _gate_body(x_ref, wt_ref, gb_ref, bg_ref, eidx_ref, w2_ref):
    xb = x_ref[...] - bg_ref[...]
    logits = jax.lax.dot_general(
        xb, wt_ref[...], (((1,), (0,)), ((), ())),
        preferred_element_type=jnp.float32, precision=_HI) + gb_ref[...]
    p = jax.nn.softmax(logits, axis=-1)
    iota = jax.lax.broadcasted_iota(jnp.int32, p.shape, 1)
    m1 = jnp.max(p, axis=1, keepdims=True)
    a1 = jnp.min(jnp.where(p >= m1, iota, EXPERTS), axis=1, keepdims=True)
    p2 = jnp.where(iota == a1, -jnp.inf, p)
    m2 = jnp.max(p2, axis=1, keepdims=True)
    a2 = jnp.min(jnp.where(p2 >= m2, iota, EXPERTS), axis=1, keepdims=True)
    d = m2 - m1
    ed = jnp.exp(d)
    w1 = 1.0 / (1.0 + ed)
    w2 = ed / (1.0 + ed)
    eidx_ref[...] = jnp.concatenate([a1, a2], axis=1)
    w2_ref[...] = jnp.concatenate([w1, w2], axis=1)


def _gating(x, gate_W, gate_b, b_gate):
    grid = (N_TOK // GATE_BM,)
    return pl.pallas_call(
        _gate_body,
        grid=grid,
        in_specs=[
            pl.BlockSpec((GATE_BM, ACT_DIM), lambda i: (i, 0)),
            pl.BlockSpec((ACT_DIM, EXPERTS), lambda i: (0, 0)),
            pl.BlockSpec((1, EXPERTS), lambda i: (0, 0)),
            pl.BlockSpec((1, ACT_DIM), lambda i: (0, 0)),
        ],
        out_specs=[
            pl.BlockSpec((GATE_BM, E_ROUTED), lambda i: (i, 0)),
            pl.BlockSpec((GATE_BM, E_ROUTED), lambda i: (i, 0)),
        ],
        out_shape=[
            jax.ShapeDtypeStruct((N_TOK, E_ROUTED), jnp.int32),
            jax.ShapeDtypeStruct((N_TOK, E_ROUTED), jnp.float32),
        ],
    )(x, gate_W.T, gate_b[None, :], b_gate[None, :])


def _dispatch(eidx, w2):
    e_flat = eidx.reshape(-1)
    w_flat = w2.reshape(-1)
    onehot = (e_flat[:, None] ==
              jnp.arange(EXPERTS, dtype=e_flat.dtype)[None, :]).astype(jnp.int32)
    csum = jnp.cumsum(onehot, axis=0)
    rank = jnp.sum(onehot * csum, axis=1) - 1
    counts = csum[-1]
    padded = ((counts + BM - 1) // BM) * BM
    seg_pad = jnp.concatenate([jnp.zeros((1,), jnp.int32),
                               jnp.cumsum(padded)[:-1].astype(jnp.int32)])
    dest = seg_pad[e_flat] + rank
    ar = jnp.arange(e_flat.shape[0], dtype=jnp.int32)
    tok_ids = jnp.zeros((M_PAD,), jnp.int32).at[dest].set(ar // E_ROUTED)
    w_sorted = jnp.zeros((M_PAD,), jnp.float32).at[dest].set(w_flat)
    blk_expert = jnp.searchsorted(
        jnp.cumsum(padded), jnp.arange(NBLK, dtype=jnp.int32) * BM,
        side='right').astype(jnp.int32)
    blk_expert = jnp.minimum(blk_expert, EXPERTS - 1)
    return tok_ids, w_sorted, dest, blk_expert


def _gemm_body(be_ref, xs_ref, w_ref, encw_ref, encb_ref, bdec_ref, zs_ref):
    xb = xs_ref[...] - bdec_ref[...]
    z = jax.lax.dot_general(
        xb, encw_ref[...], (((1,), (1,)), ((), ())),
        preferred_element_type=jnp.float32, precision=_HI)
    z = jnp.maximum(z + encb_ref[...], 0.0) * w_ref[...]
    zs_ref[...] = z


def _grouped_gemm(xs, w_sorted, blk_expert, enc_W, enc_b, b_dec):
    grid_spec = pltpu.PrefetchScalarGridSpec(
        num_scalar_prefetch=1,
        grid=(NBLK,),
        in_specs=[
            pl.BlockSpec((BM, ACT_DIM), lambda i, be: (i, 0)),
            pl.BlockSpec((BM, 1), lambda i, be: (i, 0)),
            pl.BlockSpec((None, EXP_DICT, ACT_DIM), lambda i, be: (be[i], 0, 0)),
            pl.BlockSpec((None, 1, EXP_DICT), lambda i, be: (be[i], 0, 0)),
            pl.BlockSpec((1, ACT_DIM), lambda i, be: (0, 0)),
        ],
        out_specs=pl.BlockSpec((BM, EXP_DICT), lambda i, be: (i, 0)),
    )
    return pl.pallas_call(
        _gemm_body,
        grid_spec=grid_spec,
        out_shape=jax.ShapeDtypeStruct((M_PAD, EXP_DICT), jnp.float32),
    )(blk_expert, xs, w_sorted[:, None], enc_W, enc_b[:, None, :],
      b_dec[None, :])


THR_BM = 256


def _thr_body(cs_ref, thr_ref):
    v = cs_ref[...]
    lo = jnp.full((THR_BM, 1), -1, jnp.int32)
    hi = jnp.full((THR_BM, 1), 0x7F7FFFFF, jnp.int32)

    def step(_, carry):
        lo, hi = carry
        mid = lo + (hi - lo) // 2
        cnt = jnp.sum((v > mid).astype(jnp.int32), axis=1, keepdims=True)
        small = cnt < K
        return jnp.where(small, lo, mid + 1), jnp.where(small, mid, hi)

    lo, hi = jax.lax.fori_loop(0, 31, step, (lo, hi))
    thr_ref[...] = hi.reshape(1, THR_BM)


def _thresholds(cs_bits):
    n = N_TOK // THR_BM
    out = pl.pallas_call(
        _thr_body,
        grid=(n,),
        in_specs=[pl.BlockSpec((THR_BM, E_ROUTED * EXP_DICT), lambda i: (i, 0))],
        out_specs=pl.BlockSpec((None, 1, THR_BM), lambda i: (i, 0, 0)),
        out_shape=jax.ShapeDtypeStruct((n, 1, THR_BM), jnp.int32),
    )(cs_bits)
    return out.reshape(N_TOK)


def _sc_gather_rows(table, idx, D, CH):
    B = idx.shape[0]
    bpw = B // NW
    nch = bpw // CH
    mesh = plsc.VectorSubcoreMesh(**SC_MESH)

    @functools.partial(
        pl.kernel,
        out_type=jax.ShapeDtypeStruct((B, D), table.dtype),
        mesh=mesh,
        scratch_types=[
            pltpu.VMEM((bpw,), jnp.int32),
            pltpu.VMEM((CH, D), table.dtype),
            pltpu.SemaphoreType.DMA,
        ],
        compiler_params=pltpu.CompilerParams(needs_layout_passes=False),
    )
    def k(table_h, idx_h, out_h, idx_v, buf_v, sem):
        wid = lax.axis_index("s") * 2 + lax.axis_index("c")
        base = wid * bpw
        pltpu.sync_copy(idx_h.at[pl.ds(base, bpw)], idx_v)

        def chunk(c, carry):
            pltpu.async_copy(
                table_h.at[idx_v.at[pl.ds(c * CH, CH)]], buf_v, sem).wait()
            pltpu.sync_copy(buf_v, out_h.at[pl.ds(base + c * CH, CH)])
            return carry

        lax.fori_loop(0, nch, chunk, 0)

    return k(table, idx)


TPW = N_TOK // NW
NCH = (E_ROUTED * EXP_DICT) // 16
ND = ACT_DIM // 16


def _sc_select_decode(cs, thr, base_flat, decoder, b_dec):
    mesh = plsc.VectorSubcoreMesh(**SC_MESH)

    @functools.partial(
        pl.kernel,
        out_type=jax.ShapeDtypeStruct((N_TOK, ACT_DIM), jnp.float32),
        mesh=mesh,
        scratch_types=[
            pltpu.VMEM((2, E_ROUTED * EXP_DICT), jnp.float32),
            pltpu.VMEM((TPW,), jnp.int32),
            pltpu.VMEM((TPW * E_ROUTED,), jnp.int32),
            pltpu.VMEM((2, K), jnp.float32),
            pltpu.VMEM((2, K), jnp.int32),
            pltpu.VMEM((2, K, ACT_DIM), jnp.float32),
            pltpu.VMEM((ACT_DIM,), jnp.float32),
            pltpu.VMEM((ACT_DIM,), jnp.float32),
            pltpu.SemaphoreType.DMA((2,)),
            pltpu.SemaphoreType.DMA((2,)),
        ],
        compiler_params=pltpu.CompilerParams(needs_layout_passes=False),
    )
    def k(cs_h, thr_h, base_h, dec_h, bdec_h, out_h, csv, thrv, basev, valsv,
          idxsv, rowsv, bdecv, outv, cssem, rowsem):
        wid = lax.axis_index("s") * 2 + lax.axis_index("c")
        pltpu.sync_copy(thr_h.at[pl.ds(wid * TPW, TPW)], thrv)
        pltpu.sync_copy(base_h.at[pl.ds(wid * TPW * E_ROUTED,
                                        TPW * E_ROUTED)], basev)
        pltpu.sync_copy(bdec_h, bdecv)
        lanes = lax.iota(jnp.int32, 16)

        def cs_dma(j, s):
            return pltpu.make_async_copy(
                cs_h.at[wid * TPW + j], csv.at[s], cssem.at[s])

        def row_dma(s):
            return pltpu.make_async_copy(
                dec_h.at[idxsv.at[s]], rowsv.at[s], rowsem.at[s])

        def select(j, s):
            for b in range(K // 16):
                valsv[s, pl.ds(b * 16, 16)] = jnp.zeros((16,), jnp.float32)
                idxsv[s, pl.ds(b * 16, 16)] = jnp.zeros((16,), jnp.int32)
            tvec = plsc.load_gather(thrv, [jnp.zeros((16,), jnp.int32) + j])
            b0 = plsc.load_gather(basev, [jnp.zeros((16,), jnp.int32) + 2 * j])
            b1 = plsc.load_gather(basev,
                                  [jnp.zeros((16,), jnp.int32) + 2 * j + 1])

            def group(g, off):
                vfs, vis, masks, cnts, css = [], [], [], [], []
                for u in range(4):
                    vf = csv[s, pl.ds(g * 64 + u * 16, 16)]
                    vi = plsc.bitcast(vf, jnp.int32)
                    mask = (vi >= tvec) & (vi > 0)
                    vfs.append(vf)
                    vis.append(vi)
                    masks.append(mask)
                    cnts.append(jnp.sum(mask.astype(jnp.int32)))
                    css.append(plsc.cumsum(mask.astype(jnp.int32)))

                @pl.when(off < K)
                def _():
                    o = off
                    for u in range(4):
                        c = g * 4 + u
                        sh = (c >= NCH // 2).astype(jnp.int32)
                        gidx = b0 + sh * (b1 - b0) + (
                            lanes + c * 16 - sh * (EXP_DICT * E_ROUTED // 2))
                        pos = o + css[u] - 1
                        mask2 = masks[u] & (pos < K)
                        plsc.store_scatter(valsv.at[s], [pos], vfs[u],
                                           mask=mask2)
                        plsc.store_scatter(idxsv.at[s], [pos], gidx,
                                           mask=mask2)
                        o = o + cnts[u]

                return off + cnts[0] + cnts[1] + cnts[2] + cnts[3]

            lax.fori_loop(0, NCH // 4, group, jnp.int32(0))

        def accum_store(j, s):
            def accum(kk, acc):
                w = plsc.load_gather(valsv.at[s],
                                     [jnp.zeros((16,), jnp.int32) + kk])
                return tuple(acc[d] + w * rowsv[s, kk, pl.ds(d * 16, 16)]
                             for d in range(ND))

            acc0 = tuple(bdecv[pl.ds(d * 16, 16)] for d in range(ND))
            acc = lax.fori_loop(0, K, accum, acc0)
            for d in range(ND):
                outv[pl.ds(d * 16, 16)] = acc[d]
            pltpu.sync_copy(outv, out_h.at[wid * TPW + j])

        def phase(j, s):
            cs_dma(j, s).wait()
            select(j, s)
            row_dma(s).start()

            @pl.when(j >= 1)
            def _():
                row_dma(1 - s).wait()
                accum_store(j - 1, 1 - s)

            @pl.when(j + 2 < TPW)
            def _():
                cs_dma(j + 2, s).start()

        cs_dma(0, 0).start()
        cs_dma(1, 1).start()

        def pair(jj, carry):
            phase(2 * jj, 0)
            phase(2 * jj + 1, 1)
            return carry

        lax.fori_loop(0, TPW // 2, pair, 0)
        row_dma(1).wait()
        accum_store(TPW - 1, 1)

    return k(cs, thr, base_flat, decoder, b_dec)


def kernel(x, gate_W, gate_b, enc_W, enc_b, decoder, b_gate, b_dec):
    eidx, w2 = _gating(x, gate_W, gate_b, b_gate)
    tok_ids, w_sorted, pos_flat, blk_expert = _dispatch(eidx, w2)
    xs = _sc_gather_rows(x, tok_ids, ACT_DIM, 96)
    zs = _grouped_gemm(xs, w_sorted, blk_expert, enc_W, enc_b, b_dec)
    cs = _sc_gather_rows(zs, pos_flat, EXP_DICT, 64)
    cs = cs.reshape(N_TOK, E_ROUTED * EXP_DICT)
    thr = _thresholds(cs.view(jnp.int32))
    base_flat = (eidx * EXP_DICT).reshape(-1)
    return _sc_select_decode(cs, thr, base_flat, decoder, b_dec)
